# Initial kernel scaffold; baseline (speedup 1.0000x reference)
import jax
import jax.numpy as jnp
from jax.experimental import pallas as pl


def kernel(features, pos, proj_w1, proj_b1, proj_w2, proj_b2, off_w, off_b, wo_w, wo_b, wl_w, wl_b, next_w, next_b, ln_g, ln_b):
    raise NotImplementedError("write your pallas kernel here")


# speedup vs baseline: 15.6150x; 15.6150x over previous
import functools

import jax
import jax.numpy as jnp
import numpy as np
from jax import lax
from jax.experimental import pallas as pl
from jax.experimental.pallas import tpu as pltpu
from jax.experimental.pallas import tpu_sc as plsc

B, C, T, H, W, L = 2, 256, 8, 24, 24, 4
THW = T * H * W
BL = B * L
NPTS = 4
N = NPTS ** 3
NO = 4
NK = 32
NROWS = BL * N * NK

_ti = np.round(np.linspace(T / NPTS / 2, T - 1 - T / NPTS / 2, NPTS)).astype(np.int32)
_hi = np.round(np.linspace(H / NPTS / 2, H - 1 - H / NPTS / 2, NPTS)).astype(np.int32)
_wi = np.round(np.linspace(W / NPTS / 2, W - 1 - W / NPTS / 2, NPTS)).astype(np.int32)
_tt = np.broadcast_to(_ti[:, None, None], (NPTS,) * 3).reshape(-1)
_hh = np.broadcast_to(_hi[None, :, None], (NPTS,) * 3).reshape(-1)
_ww = np.broadcast_to(_wi[None, None, :], (NPTS,) * 3).reshape(-1)
_V = (H * W * _tt + W * _hh + _ww).astype(np.int32)

_thw = np.stack([_tt / (T - 1), _hh / (H - 1), _ww / (W - 1)], -1)
_thw = np.clip(_thw, 0.0, 1.0)
_IST = np.log(np.clip(_thw, 1e-5, None) / np.clip(1.0 - _thw, 1e-5, None)).astype(np.float32)


def _sigmoid(x):
    return 1.0 / (1.0 + jnp.exp(-x))


def _stage_a(f_ref, p_ref, w1t_ref, b1_ref, w2t_ref, b2_ref, hw_ref, hb_ref,
             ist_ref, out_ref):
    fb = f_ref[0]
    glob = jnp.sum(fb, axis=0, keepdims=True) * (1.0 / THW)
    f_s = jnp.concatenate([fb[v:v + 1, :] for v in _V.tolist()], axis=0)
    x = jnp.concatenate([f_s, p_ref[0], jnp.broadcast_to(glob, (N, C))], axis=1)
    h1 = jnp.maximum(jnp.dot(x, w1t_ref[...], preferred_element_type=jnp.float32)
                     + b1_ref[...], 0.0)
    src = jnp.maximum(jnp.dot(h1, w2t_ref[...], preferred_element_type=jnp.float32)
                      + b2_ref[...], 0.0)
    heads = jnp.dot(src, hw_ref[...], preferred_element_type=jnp.float32) + hb_ref[...]
    nxt = heads[:, 0:3]
    wl_log = heads[:, 3:7]
    st = heads[:, 7:11]
    sh = heads[:, 11:15]
    sw = heads[:, 15:19]
    wo = _sigmoid(heads[:, 19:23])

    e = jnp.exp(wl_log - jnp.max(wl_log, axis=1, keepdims=True))
    s_l = jnp.sum(e / jnp.sum(e, axis=1, keepdims=True), axis=1, keepdims=True)

    ist = ist_ref[...]
    s0 = _sigmoid(ist[:, 0:1] + st) * 2.0 - 1.0
    s1 = _sigmoid(ist[:, 1:2] + sh) * 2.0 - 1.0
    s2 = _sigmoid(ist[:, 2:3] + sw) * 2.0 - 1.0
    ix = ((s0 + 1.0) * W - 1.0) * 0.5
    iy = ((s1 + 1.0) * H - 1.0) * 0.5
    iz = ((s2 + 1.0) * T - 1.0) * 0.5
    ix0 = jnp.floor(ix)
    iy0 = jnp.floor(iy)
    iz0 = jnp.floor(iz)
    fx = ix - ix0
    fy = iy - iy0
    fz = iz - iz0

    base = pl.program_id(0) * THW
    cw_cols = []
    ci_cols = []
    for dz in (0, 1):
        for dy in (0, 1):
            for dx in (0, 1):
                xi = ix0 + dx
                yi = iy0 + dy
                zi = iz0 + dz
                wgt = ((fx if dx else 1.0 - fx) * (fy if dy else 1.0 - fy)
                       * (fz if dz else 1.0 - fz))
                valid = ((xi >= 0) & (xi <= W - 1) & (yi >= 0) & (yi <= H - 1)
                         & (zi >= 0) & (zi <= T - 1))
                xc = jnp.clip(xi, 0, W - 1).astype(jnp.int32)
                yc = jnp.clip(yi, 0, H - 1).astype(jnp.int32)
                zc = jnp.clip(zi, 0, T - 1).astype(jnp.int32)
                fl = (zc * H + yc) * W + xc + base
                cw_cols.append(wgt * valid.astype(jnp.float32) * wo)
                ci_cols.append(fl.astype(jnp.float32))
    cw = jnp.concatenate(cw_cols, axis=1)
    ci = jnp.concatenate(ci_cols, axis=1)

    disp = _sigmoid(ist[:, 0:3] + nxt)
    d0 = jnp.round(disp[:, 0:1] * (T - 1.0))
    d1 = jnp.round(disp[:, 1:2] * (H - 1.0))
    d2 = jnp.round(disp[:, 2:3] * (W - 1.0))
    ni = (H * W) * d0 + W * d1 + d2

    out_ref[0] = jnp.concatenate(
        [f_s, cw, ci, s_l, ni, jnp.zeros((N, 512 - C - 2 * NK - 2), jnp.float32)], axis=1)


def _stage_c(g_ref, a_ref, lng_ref, lnb_ref, out_ref):
    a = a_ref[0]
    acc = a[:, 0:C]
    for kk in range(NK):
        acc = acc + g_ref[0, kk * N:(kk + 1) * N, :] * a[:, C + kk:C + kk + 1]
    v = acc * a[:, C + 2 * NK:C + 2 * NK + 1]
    mu = jnp.mean(v, axis=1, keepdims=True)
    var = jnp.mean((v - mu) ** 2, axis=1, keepdims=True)
    ln = (v - mu) * jax.lax.rsqrt(var + 1e-5) * lng_ref[...] + lnb_ref[...]
    out_ref[0] = jnp.broadcast_to(lnb_ref[...], (THW, C))
    for n, vx in enumerate(_V.tolist()):
        out_ref[0, vx:vx + 1, :] = ln[n:n + 1, :]


@functools.lru_cache(maxsize=1)
def _sc_gather_build():
    info = plsc.get_sparse_core_info()
    nw = info.num_cores * info.num_subcores
    rpw = NROWS // nw
    chunk = 128
    mesh = plsc.VectorSubcoreMesh(core_axis_name="c", subcore_axis_name="s")

    @functools.partial(
        pl.kernel,
        out_type=jax.ShapeDtypeStruct((NROWS, C), jnp.float32),
        mesh=mesh,
        scratch_types=[
            pltpu.VMEM((rpw,), jnp.int32),
            pltpu.VMEM((chunk, C), jnp.float32),
            pltpu.SemaphoreType.DMA,
        ],
    )
    def sc_gather(table_hbm, idx_hbm, out_hbm, idx_v, buf, sem):
        wid = lax.axis_index("s") * info.num_cores + lax.axis_index("c")
        base = wid * rpw
        pltpu.sync_copy(idx_hbm.at[pl.ds(base, rpw)], idx_v)
        for j in range(rpw // chunk):
            pltpu.async_copy(
                table_hbm.at[idx_v.at[pl.ds(j * chunk, chunk)]], buf, sem).wait()
            pltpu.sync_copy(buf, out_hbm.at[pl.ds(base + j * chunk, chunk)])

    return sc_gather


def _gather_rows(table, idx):
    return _sc_gather_build()(table, idx)


def kernel(features, pos, proj_w1, proj_b1, proj_w2, proj_b2, off_w, off_b,
           wo_w, wo_b, wl_w, wl_b, next_w, next_b, ln_g, ln_b):
    f_t = (features.reshape(B, C, THW, L).transpose(0, 3, 2, 1)
           .reshape(BL, THW, C))
    p_s = (pos.reshape(B, C, THW, L)[:, :, _V, :].transpose(0, 3, 2, 1)
           .reshape(BL, N, C))

    eye = jnp.eye(NO, dtype=jnp.float32)
    dd = C // NO

    def blockdiag(row):
        return (eye[:, None, :] * row[None, :, None]).reshape(C, NO)

    hw = jnp.concatenate(
        [next_w.T, wl_w.T, blockdiag(off_w[0]), blockdiag(off_w[1]),
         blockdiag(off_w[2]), blockdiag(wo_w[1] - wo_w[0]),
         jnp.zeros((C, 128 - 23), jnp.float32)], axis=1)
    hb = jnp.concatenate(
        [next_b, wl_b, jnp.full((NO,), off_b[0]), jnp.full((NO,), off_b[1]),
         jnp.full((NO,), off_b[2]), jnp.full((NO,), wo_b[1] - wo_b[0]),
         jnp.zeros((128 - 23,), jnp.float32)])[None, :]
    ist = jnp.concatenate(
        [jnp.asarray(_IST), jnp.zeros((N, 125), jnp.float32)], axis=1)

    out_a = pl.pallas_call(
        _stage_a,
        grid=(BL,),
        in_specs=[
            pl.BlockSpec((1, THW, C), lambda i: (i, 0, 0)),
            pl.BlockSpec((1, N, C), lambda i: (i, 0, 0)),
            pl.BlockSpec((3 * C, C), lambda i: (0, 0)),
            pl.BlockSpec((1, C), lambda i: (0, 0)),
            pl.BlockSpec((C, C), lambda i: (0, 0)),
            pl.BlockSpec((1, C), lambda i: (0, 0)),
            pl.BlockSpec((C, 128), lambda i: (0, 0)),
            pl.BlockSpec((1, 128), lambda i: (0, 0)),
            pl.BlockSpec((N, 128), lambda i: (0, 0)),
        ],
        out_specs=pl.BlockSpec((1, N, 512), lambda i: (i, 0, 0)),
        out_shape=jax.ShapeDtypeStruct((BL, N, 512), jnp.float32),
    )(f_t, p_s, proj_w1.T, proj_b1[None, :], proj_w2.T, proj_b2[None, :],
      hw, hb, ist)

    idx = (out_a[:, :, C + NK:C + 2 * NK].astype(jnp.int32)
           .transpose(0, 2, 1).reshape(NROWS))
    gathered = _gather_rows(f_t.reshape(BL * THW, C), idx)

    out_t = pl.pallas_call(
        _stage_c,
        grid=(BL,),
        in_specs=[
            pl.BlockSpec((1, N * NK, C), lambda i: (i, 0, 0)),
            pl.BlockSpec((1, N, 512), lambda i: (i, 0, 0)),
            pl.BlockSpec((1, C), lambda i: (0, 0)),
            pl.BlockSpec((1, C), lambda i: (0, 0)),
        ],
        out_specs=pl.BlockSpec((1, THW, C), lambda i: (i, 0, 0)),
        out_shape=jax.ShapeDtypeStruct((BL, THW, C), jnp.float32),
    )(gathered.reshape(BL, N * NK, C), out_a, ln_g[None, :], ln_b[None, :])

    feats4 = (out_t.reshape(B, L, THW, C).transpose(0, 3, 2, 1)
              .reshape(B, C, T, H, W, L))
    next_ind = (out_a[:, :, C + 2 * NK + 1].astype(jnp.int32)
                .reshape(B, L, N))
    return feats4, next_ind
